# baseline (speedup 1.0000x reference)
import functools

import jax
import jax.numpy as jnp
from jax import lax
from jax.experimental import pallas as pl
from jax.experimental.pallas import tpu as pltpu
from jax.experimental.pallas import tpu_sc as plsc

NN = 10000
EE = 320000
DD = 128
HH = 128
PP = 64

NC = 2
NS = 16
K = 80
CH = EE // K
CHC = CH // NC
IPT = CHC // NS
NB = 2
NP = 10240
RPT = NP // NS
RCH = 128
DW = 8
BN_C = 1.0 / (1.0 + 1e-5) ** 0.5


def _make_sc_agg(w, with_deg):
    mesh = plsc.VectorSubcoreMesh(core_axis_name="c", subcore_axis_name="s")

    out_type = jax.ShapeDtypeStruct((NC, NP, w), jnp.float32)
    if with_deg:
        out_type = [out_type, jax.ShapeDtypeStruct((NC, NP, DW), jnp.float32)]
    scratch = [
        pltpu.VMEM((IPT, K), jnp.int32),
        pltpu.VMEM((IPT, K), jnp.int32),
        [pltpu.VMEM((K, w), jnp.float32)] * NB,
        pltpu.VMEM_SHARED((NP, w), jnp.float32),
        pltpu.SemaphoreType.DMA,
        [pltpu.SemaphoreType.DMA] * NB,
    ]
    if with_deg:
        scratch += [
            pltpu.VMEM((K, DW), jnp.float32),
            pltpu.VMEM((RCH, DW), jnp.float32),
            pltpu.VMEM_SHARED((NP, DW), jnp.float32),
        ]

    @functools.partial(
        pl.kernel,
        out_type=out_type,
        mesh=mesh,
        compiler_params=pltpu.CompilerParams(use_tc_tiling_on_sc=False),
        scratch_types=scratch,
    )
    def agg(z_hbm, src_hbm, dst_hbm, *refs):
        if with_deg:
            (out_hbm, outd_hbm, srcs_v, dsts_v, rows, acc,
             isem, gsems, ones_v, zdeg, accd) = refs
        else:
            (out_hbm, srcs_v, dsts_v, rows, acc, isem, gsems) = refs
        c = lax.axis_index("c")
        s = lax.axis_index("s")

        row0 = c * CHC + s * IPT
        di = pltpu.async_copy(src_hbm.at[pl.ds(row0, IPT)], srcs_v, isem)
        dj = pltpu.async_copy(dst_hbm.at[pl.ds(row0, IPT)], dsts_v, isem)

        zv = jnp.zeros((16,), jnp.float32)

        def zrow(i, carry):
            for j in range(w // 16):
                rows[0][i, pl.ds(j * 16, 16)] = zv
            return carry

        lax.fori_loop(0, K, zrow, 0)
        offs = list(range(0, RPT - K + 1, K))
        if offs[-1] != RPT - K:
            offs.append(RPT - K)
        for o in offs:
            pltpu.sync_copy(rows[0], acc.at[pl.ds(s * RPT + o, K)])

        if with_deg:
            ov = jnp.ones((16,), jnp.float32)

            def drow(i, carry):
                zdeg[i, pl.ds(0, 16)] = zv
                return carry

            lax.fori_loop(0, RCH, drow, 0)
            for j in range(RPT // RCH):
                pltpu.sync_copy(zdeg, accd.at[pl.ds(s * RPT + j * RCH, RCH)])
            for i in range(K):
                ones_v[i, pl.ds(0, 16)] = ov

        di.wait()
        dj.wait()
        for b in range(NB):
            pltpu.async_copy(z_hbm.at[srcs_v.at[b]], rows[b], gsems[b])
        plsc.subcore_barrier()

        def wait_gather(b):
            pltpu.make_async_copy(z_hbm.at[pl.ds(0, K)], rows[b],
                                  gsems[b]).wait()

        def chunk_deg(idx):
            if with_deg:
                pltpu.sync_copy(ones_v, accd.at[dsts_v.at[idx]], add=True)

        def step(i, carry):
            for b in range(NB):
                idx = NB * i + b
                wait_gather(b)
                chunk_deg(idx)
                pltpu.sync_copy(rows[b], acc.at[dsts_v.at[idx]], add=True)
                pltpu.async_copy(z_hbm.at[srcs_v.at[NB * (i + 1) + b]],
                                 rows[b], gsems[b])
            return carry

        fl = (IPT - NB) // NB
        lax.fori_loop(0, fl, step, 0)
        for b in range(NB):
            idx = NB * fl + b
            wait_gather(b)
            chunk_deg(idx)
            pltpu.sync_copy(rows[b], acc.at[dsts_v.at[idx]], add=True)
        for t in range(IPT - NB * fl - NB):
            idx = NB * fl + NB + t
            pltpu.async_copy(z_hbm.at[srcs_v.at[idx]], rows[t], gsems[t])
            wait_gather(t)
            chunk_deg(idx)
            pltpu.sync_copy(rows[t], acc.at[dsts_v.at[idx]], add=True)
        plsc.subcore_barrier()

        for j in range(RPT // RCH):
            r0 = s * RPT + j * RCH
            pltpu.sync_copy(acc.at[pl.ds(r0, RCH)],
                            out_hbm.at[c, pl.ds(r0, RCH)])
        if with_deg:
            pltpu.sync_copy(accd.at[pl.ds(s * RPT, RPT)],
                            outd_hbm.at[c, pl.ds(s * RPT, RPT)])

    return agg


_sc_agg_cache = {}


def _sc_agg(w, with_deg=False):
    key = (w, with_deg)
    if key not in _sc_agg_cache:
        _sc_agg_cache[key] = _make_sc_agg(w, with_deg)
    return _sc_agg_cache[key]


def _unpack_half(x, w):
    f = 128 // w
    parts = [x[:, i * w:(i + 1) * w] for i in range(f)]
    return jnp.stack(parts, axis=1).reshape(x.shape[0] * f, w)


def _tc_pre_body(x_ref, wl_ref, wr_ref, bl_ref, z_ref, r_ref):
    x = x_ref[...]
    z_ref[...] = lax.dot_general(x, wl_ref[...], (((1,), (1,)), ((), ())),
                                 preferred_element_type=jnp.float32)
    r_ref[...] = lax.dot_general(x, wr_ref[...], (((1,), (1,)), ((), ())),
                                 preferred_element_type=jnp.float32) + bl_ref[...][None, :]


def _tc_mid1_body(sp_ref, dp_ref, r_ref, g_ref, b_ref, wl_ref, wr_ref, bl_ref,
                  z_ref, rn_ref, invd_ref):
    sp = sp_ref[0] + sp_ref[1]
    dsum = dp_ref[0] + dp_ref[1]
    deg = _unpack_half(dsum, DW)[:, 0:1]
    invd = 1.0 / jnp.maximum(deg, 1.0)
    h = sp * invd + r_ref[...]
    h = jnp.maximum(h * (BN_C * g_ref[...])[None, :] + b_ref[...][None, :], 0.0)
    z_ref[...] = lax.dot_general(h, wl_ref[...], (((1,), (1,)), ((), ())),
                                 preferred_element_type=jnp.float32)
    rn_ref[...] = lax.dot_general(h, wr_ref[...], (((1,), (1,)), ((), ())),
                                  preferred_element_type=jnp.float32) + bl_ref[...][None, :]
    invd_ref[...] = invd


def _tc_mid2_body(sp_ref, r_ref, invd_ref, g_ref, b_ref, wl_ref, wr_ref, bl_ref,
                  z_ref, rn_ref):
    sp = sp_ref[0] + sp_ref[1]
    invd = invd_ref[...]
    h = sp * invd + r_ref[...]
    h = jnp.maximum(h * (BN_C * g_ref[...])[None, :] + b_ref[...][None, :], 0.0)
    z_ref[...] = lax.dot_general(h, wl_ref[...], (((1,), (1,)), ((), ())),
                                 preferred_element_type=jnp.float32)
    rn_ref[...] = lax.dot_general(h, wr_ref[...], (((1,), (1,)), ((), ())),
                                  preferred_element_type=jnp.float32) + bl_ref[...][None, :]


def _tc_fin_body(sp_ref, r_ref, invd_ref, out_ref):
    o = (sp_ref[0] + sp_ref[1]) * invd_ref[...] + r_ref[...]
    m = jnp.max(o, axis=1, keepdims=True)
    e = jnp.exp(o - m)
    out_ref[...] = e / jnp.sum(e, axis=1, keepdims=True)


_f32 = jnp.float32
BR = 2048
TG = -(-NN // BR)


def _rowspec(cols):
    return pl.BlockSpec((BR, cols), lambda i: (i, 0))


def _spspec(w):
    return pl.BlockSpec((NC, BR, w), lambda i: (0, i, 0))


def _packspec(w):
    return pl.BlockSpec((NC, BR * w // 128, 128), lambda i: (0, i, 0))


def _fullspec(shape):
    nd = len(shape)
    return pl.BlockSpec(shape, lambda i, _n=nd: (0,) * _n)


_tc_pre = pl.pallas_call(
    _tc_pre_body,
    grid=(TG,),
    in_specs=[_rowspec(DD), _fullspec((HH, DD)), _fullspec((HH, DD)),
              _fullspec((HH,))],
    out_specs=[_rowspec(HH), _rowspec(HH)],
    out_shape=[jax.ShapeDtypeStruct((NN, HH), _f32),
               jax.ShapeDtypeStruct((NN, HH), _f32)],
)

_tc_mid1 = pl.pallas_call(
    _tc_mid1_body,
    grid=(TG,),
    in_specs=[_spspec(HH), _packspec(DW), _rowspec(HH),
              _fullspec((HH,)), _fullspec((HH,)),
              _fullspec((HH, HH)), _fullspec((HH, HH)), _fullspec((HH,))],
    out_specs=[_rowspec(HH), _rowspec(HH), _rowspec(1)],
    out_shape=[jax.ShapeDtypeStruct((NN, HH), _f32),
               jax.ShapeDtypeStruct((NN, HH), _f32),
               jax.ShapeDtypeStruct((NN, 1), _f32)],
)

_tc_mid2 = pl.pallas_call(
    _tc_mid2_body,
    grid=(TG,),
    in_specs=[_spspec(HH), _rowspec(HH), _rowspec(1),
              _fullspec((HH,)), _fullspec((HH,)),
              _fullspec((PP, HH)), _fullspec((PP, HH)), _fullspec((PP,))],
    out_specs=[_rowspec(PP), _rowspec(PP)],
    out_shape=[jax.ShapeDtypeStruct((NN, PP), _f32),
               jax.ShapeDtypeStruct((NN, PP), _f32)],
)

_tc_fin = pl.pallas_call(
    _tc_fin_body,
    grid=(TG,),
    in_specs=[_spspec(PP), _rowspec(PP), _rowspec(1)],
    out_specs=_rowspec(PP),
    out_shape=jax.ShapeDtypeStruct((NN, PP), _f32),
)


def kernel(x, edge_index, Wl0, bl0, Wr0, g1, b1, Wl1, bl1, Wr1, g2, b2, Wl2,
           bl2, Wr2):
    src = edge_index[0].reshape(CH, K)
    dst = edge_index[1].reshape(CH, K)

    def _v(a):
        return a.reshape(NC, NP * DW // 128, 128)

    z0, r0 = _tc_pre(x, Wl0, Wr0, bl0)
    s0, degp = _sc_agg(HH, True)(z0, src, dst)
    z1, r1, invd = _tc_mid1(s0, _v(degp), r0, g1, b1, Wl1, Wr1, bl1)
    s1 = _sc_agg(HH)(z1, src, dst)
    z2, r2 = _tc_mid2(s1, r1, invd, g2, b2, Wl2, Wr2, bl2)
    s2 = _sc_agg(PP)(z2, src, dst)
    return _tc_fin(s2, r2, invd)


# speedup vs baseline: 1.1189x; 1.1189x over previous
import functools

import jax
import jax.numpy as jnp
from jax import lax
from jax.experimental import pallas as pl
from jax.experimental.pallas import tpu as pltpu
from jax.experimental.pallas import tpu_sc as plsc

NN = 10000
EE = 320000
DD = 128
HH = 128
PP = 64

NC = 2
NS = 16
K = 125
CH = EE // K
IPT = CH // NS
NB = 4
NP = 10240
RPT = NP // NS
RCH = 128
DW = 16
BN_C = 1.0 / (1.0 + 1e-5) ** 0.5


def _make_sc_agg(w, with_deg):
    mesh = plsc.VectorSubcoreMesh(core_axis_name="c", subcore_axis_name="s")

    out_type = jax.ShapeDtypeStruct((NC, NP, w), jnp.float32)
    if with_deg:
        out_type = [out_type, jax.ShapeDtypeStruct((NC, NP, DW), jnp.float32)]
    scratch = [
        pltpu.VMEM((IPT, K), jnp.int32),
        pltpu.VMEM((IPT, K), jnp.int32),
        [pltpu.VMEM((K, w), jnp.float32)] * NB,
        pltpu.VMEM_SHARED((NP, w), jnp.float32),
        pltpu.SemaphoreType.DMA,
        [pltpu.SemaphoreType.DMA] * NB,
    ]
    if with_deg:
        scratch += [
            pltpu.VMEM((K, DW), jnp.float32),
            pltpu.VMEM((RCH, DW), jnp.float32),
            pltpu.VMEM_SHARED((NP, DW), jnp.float32),
        ]

    @functools.partial(
        pl.kernel,
        out_type=out_type,
        mesh=mesh,
        compiler_params=pltpu.CompilerParams(use_tc_tiling_on_sc=False),
        scratch_types=scratch,
    )
    def agg(z_hbm, src_hbm, dst_hbm, *refs):
        if with_deg:
            (out_hbm, outd_hbm, srcs_v, dsts_v, rows, acc,
             isem, gsems, ones_v, zdeg, accd) = refs
        else:
            (out_hbm, srcs_v, dsts_v, rows, acc, isem, gsems) = refs
        c = lax.axis_index("c")
        s = lax.axis_index("s")

        row0 = s * IPT
        di = pltpu.async_copy(src_hbm.at[c, pl.ds(row0, IPT)], srcs_v, isem)
        dj = pltpu.async_copy(dst_hbm.at[pl.ds(row0, IPT)], dsts_v, isem)

        zv = jnp.zeros((16,), jnp.float32)

        def zrow(i, carry):
            for j in range(w // 16):
                rows[0][i, pl.ds(j * 16, 16)] = zv
            return carry

        lax.fori_loop(0, K, zrow, 0)
        offs = list(range(0, RPT - K + 1, K))
        if offs[-1] != RPT - K:
            offs.append(RPT - K)
        for o in offs:
            pltpu.sync_copy(rows[0], acc.at[pl.ds(s * RPT + o, K)])

        if with_deg:
            ov = jnp.ones((16,), jnp.float32)

            def drow(i, carry):
                zdeg[i, pl.ds(0, 16)] = zv
                return carry

            lax.fori_loop(0, RCH, drow, 0)
            for j in range(RPT // RCH):
                pltpu.sync_copy(zdeg, accd.at[pl.ds(s * RPT + j * RCH, RCH)])
            for i in range(K):
                ones_v[i, pl.ds(0, 16)] = ov

        di.wait()
        dj.wait()
        for b in range(NB):
            pltpu.async_copy(z_hbm.at[srcs_v.at[b]], rows[b], gsems[b])
        plsc.subcore_barrier()

        def wait_gather(b):
            pltpu.make_async_copy(z_hbm.at[pl.ds(0, K)], rows[b],
                                  gsems[b]).wait()

        def chunk_deg(b, idx):
            if with_deg:
                @pl.when(c == (b % 2))
                def _():
                    pltpu.sync_copy(ones_v, accd.at[dsts_v.at[idx]], add=True)

        def step(i, carry):
            for b in range(NB):
                idx = NB * i + b
                wait_gather(b)
                chunk_deg(b, idx)
                pltpu.sync_copy(rows[b], acc.at[dsts_v.at[idx]], add=True)
                pltpu.async_copy(z_hbm.at[srcs_v.at[NB * (i + 1) + b]],
                                 rows[b], gsems[b])
            return carry

        lax.fori_loop(0, IPT // NB - 1, step, 0)
        for b in range(NB):
            idx = IPT - NB + b
            wait_gather(b)
            chunk_deg(b, idx)
            pltpu.sync_copy(rows[b], acc.at[dsts_v.at[idx]], add=True)
        plsc.subcore_barrier()

        for j in range(RPT // RCH):
            r0 = s * RPT + j * RCH
            pltpu.sync_copy(acc.at[pl.ds(r0, RCH)],
                            out_hbm.at[c, pl.ds(r0, RCH)])
        if with_deg:
            pltpu.sync_copy(accd.at[pl.ds(s * RPT, RPT)],
                            outd_hbm.at[c, pl.ds(s * RPT, RPT)])

    return agg


_sc_agg_cache = {}


def _sc_agg(w, with_deg=False):
    key = (w, with_deg)
    if key not in _sc_agg_cache:
        _sc_agg_cache[key] = _make_sc_agg(w, with_deg)
    return _sc_agg_cache[key]


def _unpack_half(x, w):
    f = 128 // w
    parts = [x[:, i * w:(i + 1) * w] for i in range(f)]
    return jnp.stack(parts, axis=1).reshape(x.shape[0] * f, w)


def _unpack_sp(sp_ref, w):
    return jnp.concatenate(
        [_unpack_half(sp_ref[c], w) for c in range(NC)], axis=1)


def _tc_pre_body(x_ref, wl_ref, wr_ref, bl_ref, z_ref, r_ref):
    x = x_ref[...]
    z_ref[...] = lax.dot_general(x, wl_ref[...], (((1,), (1,)), ((), ())),
                                 preferred_element_type=jnp.float32)
    r_ref[...] = lax.dot_general(x, wr_ref[...], (((1,), (1,)), ((), ())),
                                 preferred_element_type=jnp.float32) + bl_ref[...][None, :]


def _tc_mid1_body(sp_ref, dp_ref, r_ref, g_ref, b_ref, wl_ref, wr_ref, bl_ref,
                  z_ref, rn_ref, invd_ref):
    sp = _unpack_sp(sp_ref, HH // 2)
    dsum = dp_ref[0] + dp_ref[1]
    deg = _unpack_half(dsum, DW)[:, 0:1]
    invd = 1.0 / jnp.maximum(deg, 1.0)
    h = sp * invd + r_ref[...]
    h = jnp.maximum(h * (BN_C * g_ref[...])[None, :] + b_ref[...][None, :], 0.0)
    z_ref[...] = lax.dot_general(h, wl_ref[...], (((1,), (1,)), ((), ())),
                                 preferred_element_type=jnp.float32)
    rn_ref[...] = lax.dot_general(h, wr_ref[...], (((1,), (1,)), ((), ())),
                                  preferred_element_type=jnp.float32) + bl_ref[...][None, :]
    invd_ref[...] = invd


def _tc_mid2_body(sp_ref, r_ref, invd_ref, g_ref, b_ref, wl_ref, wr_ref, bl_ref,
                  z_ref, rn_ref):
    sp = _unpack_sp(sp_ref, HH // 2)
    invd = invd_ref[...]
    h = sp * invd + r_ref[...]
    h = jnp.maximum(h * (BN_C * g_ref[...])[None, :] + b_ref[...][None, :], 0.0)
    z_ref[...] = lax.dot_general(h, wl_ref[...], (((1,), (1,)), ((), ())),
                                 preferred_element_type=jnp.float32)
    rn_ref[...] = lax.dot_general(h, wr_ref[...], (((1,), (1,)), ((), ())),
                                  preferred_element_type=jnp.float32) + bl_ref[...][None, :]


def _tc_fin_body(sp_ref, r_ref, invd_ref, out_ref):
    o = _unpack_sp(sp_ref, PP // 2) * invd_ref[...] + r_ref[...]
    m = jnp.max(o, axis=1, keepdims=True)
    e = jnp.exp(o - m)
    out_ref[...] = e / jnp.sum(e, axis=1, keepdims=True)


_f32 = jnp.float32
BR = 2048
TG = -(-NN // BR)


def _rowspec(cols):
    return pl.BlockSpec((BR, cols), lambda i: (i, 0))


def _packspec(w):
    return pl.BlockSpec((NC, BR * w // 128, 128), lambda i: (0, i, 0))


def _fullspec(shape):
    nd = len(shape)
    return pl.BlockSpec(shape, lambda i, _n=nd: (0,) * _n)


_tc_pre = pl.pallas_call(
    _tc_pre_body,
    grid=(TG,),
    in_specs=[_rowspec(DD), _fullspec((HH, DD)), _fullspec((HH, DD)),
              _fullspec((HH,))],
    out_specs=[_rowspec(HH), _rowspec(HH)],
    out_shape=[jax.ShapeDtypeStruct((NN, HH), _f32),
               jax.ShapeDtypeStruct((NN, HH), _f32)],
)

_tc_mid1 = pl.pallas_call(
    _tc_mid1_body,
    grid=(TG,),
    in_specs=[_packspec(HH // 2), _packspec(DW), _rowspec(HH),
              _fullspec((HH,)), _fullspec((HH,)),
              _fullspec((HH, HH)), _fullspec((HH, HH)), _fullspec((HH,))],
    out_specs=[_rowspec(HH), _rowspec(HH), _rowspec(1)],
    out_shape=[jax.ShapeDtypeStruct((NN, HH), _f32),
               jax.ShapeDtypeStruct((NN, HH), _f32),
               jax.ShapeDtypeStruct((NN, 1), _f32)],
)

_tc_mid2 = pl.pallas_call(
    _tc_mid2_body,
    grid=(TG,),
    in_specs=[_packspec(HH // 2), _rowspec(HH), _rowspec(1),
              _fullspec((HH,)), _fullspec((HH,)),
              _fullspec((PP, HH)), _fullspec((PP, HH)), _fullspec((PP,))],
    out_specs=[_rowspec(PP), _rowspec(PP)],
    out_shape=[jax.ShapeDtypeStruct((NN, PP), _f32),
               jax.ShapeDtypeStruct((NN, PP), _f32)],
)

_tc_fin = pl.pallas_call(
    _tc_fin_body,
    grid=(TG,),
    in_specs=[_packspec(PP // 2), _rowspec(PP), _rowspec(1)],
    out_specs=_rowspec(PP),
    out_shape=jax.ShapeDtypeStruct((NN, PP), _f32),
)


def kernel(x, edge_index, Wl0, bl0, Wr0, g1, b1, Wl1, bl1, Wr1, g2, b2, Wl2,
           bl2, Wr2):
    src2 = jnp.stack([2 * edge_index[0], 2 * edge_index[0] + 1]).reshape(
        NC, CH, K)
    dst = edge_index[1].reshape(CH, K)

    def _v(a, w):
        return a.reshape(NC, NP * w // 128, 128)

    z0, r0 = _tc_pre(x, Wl0, Wr0, bl0)
    s0, degp = _sc_agg(HH // 2, True)(z0.reshape(2 * NN, HH // 2), src2, dst)
    z1, r1, invd = _tc_mid1(_v(s0, HH // 2), _v(degp, DW), r0,
                            g1, b1, Wl1, Wr1, bl1)
    s1 = _sc_agg(HH // 2)(z1.reshape(2 * NN, HH // 2), src2, dst)
    z2, r2 = _tc_mid2(_v(s1, HH // 2), r1, invd, g2, b2, Wl2, Wr2, bl2)
    s2 = _sc_agg(PP // 2)(z2.reshape(2 * NN, PP // 2), src2, dst)
    return _tc_fin(_v(s2, PP // 2), r2, invd)


# speedup vs baseline: 1.1240x; 1.0045x over previous
import functools

import jax
import jax.numpy as jnp
from jax import lax
from jax.experimental import pallas as pl
from jax.experimental.pallas import tpu as pltpu
from jax.experimental.pallas import tpu_sc as plsc

NN = 10000
EE = 320000
DD = 128
HH = 128
PP = 64

NC = 2
NS = 16
K = 125
CH = EE // K
IPT = CH // NS
NB = 4
NP = 10240
RPT = NP // NS
RCH = 128
DW = 16
BN_C = 1.0 / (1.0 + 1e-5) ** 0.5


def _make_sc_agg(w, with_deg):
    mesh = plsc.VectorSubcoreMesh(core_axis_name="c", subcore_axis_name="s")

    out_type = jax.ShapeDtypeStruct((NC, NP, w), jnp.float32)
    if with_deg:
        out_type = [out_type, jax.ShapeDtypeStruct((NC, NP, DW), jnp.float32)]
    scratch = [
        pltpu.VMEM((IPT, K), jnp.int32),
        pltpu.VMEM((IPT, K), jnp.int32),
        [pltpu.VMEM((K, w), jnp.float32)] * NB,
        pltpu.VMEM_SHARED((NP, w), jnp.float32),
        pltpu.SemaphoreType.DMA,
        [pltpu.SemaphoreType.DMA] * NB,
    ]
    if with_deg:
        scratch += [
            pltpu.VMEM((K, DW), jnp.float32),
            pltpu.VMEM((RCH, DW), jnp.float32),
            pltpu.VMEM_SHARED((NP, DW), jnp.float32),
        ]

    @functools.partial(
        pl.kernel,
        out_type=out_type,
        mesh=mesh,
        compiler_params=pltpu.CompilerParams(use_tc_tiling_on_sc=False),
        scratch_types=scratch,
    )
    def agg(z_hbm, src_hbm, dst_hbm, *refs):
        if with_deg:
            (out_hbm, outd_hbm, srcs_v, dsts_v, rows, acc,
             isem, gsems, ones_v, zdeg, accd) = refs
        else:
            (out_hbm, srcs_v, dsts_v, rows, acc, isem, gsems) = refs
        c = lax.axis_index("c")
        s = lax.axis_index("s")

        row0 = s * IPT
        di = pltpu.async_copy(src_hbm.at[c, pl.ds(row0, IPT)], srcs_v, isem)
        dj = pltpu.async_copy(dst_hbm.at[pl.ds(row0, IPT)], dsts_v, isem)

        zv = jnp.zeros((16,), jnp.float32)

        def zrow(i, carry):
            for j in range(w // 16):
                rows[0][i, pl.ds(j * 16, 16)] = zv
            return carry

        lax.fori_loop(0, K, zrow, 0)
        offs = list(range(0, RPT - K + 1, K))
        if offs[-1] != RPT - K:
            offs.append(RPT - K)
        for o in offs:
            pltpu.sync_copy(rows[0], acc.at[pl.ds(s * RPT + o, K)])

        if with_deg:
            ov = jnp.ones((16,), jnp.float32)

            def drow(i, carry):
                zdeg[i, pl.ds(0, 16)] = zv
                return carry

            lax.fori_loop(0, RCH, drow, 0)
            for j in range(RPT // RCH):
                pltpu.sync_copy(zdeg, accd.at[pl.ds(s * RPT + j * RCH, RCH)])
            for i in range(K):
                ones_v[i, pl.ds(0, 16)] = ov

        di.wait()
        dj.wait()
        for b in range(NB):
            pltpu.async_copy(z_hbm.at[srcs_v.at[b]], rows[b], gsems[b])
        plsc.subcore_barrier()

        def wait_gather(b):
            pltpu.make_async_copy(z_hbm.at[pl.ds(0, K)], rows[b],
                                  gsems[b]).wait()

        def chunk_deg(b, idx):
            if with_deg:
                @pl.when(c == (b % 2))
                def _():
                    pltpu.sync_copy(ones_v, accd.at[dsts_v.at[idx]], add=True)

        def step(i, carry):
            for b in range(NB):
                idx = NB * i + b
                wait_gather(b)
                chunk_deg(b, idx)
                pltpu.sync_copy(rows[b], acc.at[dsts_v.at[idx]], add=True)
                pltpu.async_copy(z_hbm.at[srcs_v.at[NB * (i + 1) + b]],
                                 rows[b], gsems[b])
            return carry

        lax.fori_loop(0, IPT // NB - 1, step, 0)
        for b in range(NB):
            idx = IPT - NB + b
            wait_gather(b)
            chunk_deg(b, idx)
            pltpu.sync_copy(rows[b], acc.at[dsts_v.at[idx]], add=True)
        plsc.subcore_barrier()

        pltpu.sync_copy(acc.at[pl.ds(s * RPT, RPT)],
                        out_hbm.at[c, pl.ds(s * RPT, RPT)])
        if with_deg:
            pltpu.sync_copy(accd.at[pl.ds(s * RPT, RPT)],
                            outd_hbm.at[c, pl.ds(s * RPT, RPT)])

    return agg


_sc_agg_cache = {}


def _sc_agg(w, with_deg=False):
    key = (w, with_deg)
    if key not in _sc_agg_cache:
        _sc_agg_cache[key] = _make_sc_agg(w, with_deg)
    return _sc_agg_cache[key]


def _unpack_half(x, w):
    f = 128 // w
    parts = [x[:, i * w:(i + 1) * w] for i in range(f)]
    return jnp.stack(parts, axis=1).reshape(x.shape[0] * f, w)


def _unpack_sp(sp_ref, w):
    return jnp.concatenate(
        [_unpack_half(sp_ref[c], w) for c in range(NC)], axis=1)


def _tc_pre_body(x_ref, wl_ref, wr_ref, bl_ref, z_ref, r_ref):
    x = x_ref[...]
    z_ref[...] = lax.dot_general(x, wl_ref[...], (((1,), (1,)), ((), ())),
                                 preferred_element_type=jnp.float32)
    r_ref[...] = lax.dot_general(x, wr_ref[...], (((1,), (1,)), ((), ())),
                                 preferred_element_type=jnp.float32) + bl_ref[...][None, :]


def _tc_mid1_body(sp_ref, dp_ref, r_ref, g_ref, b_ref, wl_ref, wr_ref, bl_ref,
                  z_ref, rn_ref, invd_ref):
    sp = _unpack_sp(sp_ref, HH // 2)
    dsum = dp_ref[0] + dp_ref[1]
    deg = _unpack_half(dsum, DW)[:, 0:1]
    invd = 1.0 / jnp.maximum(deg, 1.0)
    h = sp * invd + r_ref[...]
    h = jnp.maximum(h * (BN_C * g_ref[...])[None, :] + b_ref[...][None, :], 0.0)
    z_ref[...] = lax.dot_general(h, wl_ref[...], (((1,), (1,)), ((), ())),
                                 preferred_element_type=jnp.float32)
    rn_ref[...] = lax.dot_general(h, wr_ref[...], (((1,), (1,)), ((), ())),
                                  preferred_element_type=jnp.float32) + bl_ref[...][None, :]
    invd_ref[...] = invd


def _tc_mid2_body(sp_ref, r_ref, invd_ref, g_ref, b_ref, wl_ref, wr_ref, bl_ref,
                  z_ref, rn_ref):
    sp = _unpack_sp(sp_ref, HH // 2)
    invd = invd_ref[...]
    h = sp * invd + r_ref[...]
    h = jnp.maximum(h * (BN_C * g_ref[...])[None, :] + b_ref[...][None, :], 0.0)
    z_ref[...] = lax.dot_general(h, wl_ref[...], (((1,), (1,)), ((), ())),
                                 preferred_element_type=jnp.float32)
    rn_ref[...] = lax.dot_general(h, wr_ref[...], (((1,), (1,)), ((), ())),
                                  preferred_element_type=jnp.float32) + bl_ref[...][None, :]


def _tc_fin_body(sp_ref, r_ref, invd_ref, out_ref):
    o = _unpack_sp(sp_ref, PP // 2) * invd_ref[...] + r_ref[...]
    m = jnp.max(o, axis=1, keepdims=True)
    e = jnp.exp(o - m)
    out_ref[...] = e / jnp.sum(e, axis=1, keepdims=True)


_f32 = jnp.float32
BR = 2048
TG = -(-NN // BR)


def _rowspec(cols):
    return pl.BlockSpec((BR, cols), lambda i: (i, 0))


def _packspec(w):
    return pl.BlockSpec((NC, BR * w // 128, 128), lambda i: (0, i, 0))


def _fullspec(shape):
    nd = len(shape)
    return pl.BlockSpec(shape, lambda i, _n=nd: (0,) * _n)


_tc_pre = pl.pallas_call(
    _tc_pre_body,
    grid=(TG,),
    in_specs=[_rowspec(DD), _fullspec((HH, DD)), _fullspec((HH, DD)),
              _fullspec((HH,))],
    out_specs=[_rowspec(HH), _rowspec(HH)],
    out_shape=[jax.ShapeDtypeStruct((NN, HH), _f32),
               jax.ShapeDtypeStruct((NN, HH), _f32)],
)

_tc_mid1 = pl.pallas_call(
    _tc_mid1_body,
    grid=(TG,),
    in_specs=[_packspec(HH // 2), _packspec(DW), _rowspec(HH),
              _fullspec((HH,)), _fullspec((HH,)),
              _fullspec((HH, HH)), _fullspec((HH, HH)), _fullspec((HH,))],
    out_specs=[_rowspec(HH), _rowspec(HH), _rowspec(1)],
    out_shape=[jax.ShapeDtypeStruct((NN, HH), _f32),
               jax.ShapeDtypeStruct((NN, HH), _f32),
               jax.ShapeDtypeStruct((NN, 1), _f32)],
)

_tc_mid2 = pl.pallas_call(
    _tc_mid2_body,
    grid=(TG,),
    in_specs=[_packspec(HH // 2), _rowspec(HH), _rowspec(1),
              _fullspec((HH,)), _fullspec((HH,)),
              _fullspec((PP, HH)), _fullspec((PP, HH)), _fullspec((PP,))],
    out_specs=[_rowspec(PP), _rowspec(PP)],
    out_shape=[jax.ShapeDtypeStruct((NN, PP), _f32),
               jax.ShapeDtypeStruct((NN, PP), _f32)],
)

_tc_fin = pl.pallas_call(
    _tc_fin_body,
    grid=(TG,),
    in_specs=[_packspec(PP // 2), _rowspec(PP), _rowspec(1)],
    out_specs=_rowspec(PP),
    out_shape=jax.ShapeDtypeStruct((NN, PP), _f32),
)


def kernel(x, edge_index, Wl0, bl0, Wr0, g1, b1, Wl1, bl1, Wr1, g2, b2, Wl2,
           bl2, Wr2):
    src2 = jnp.stack([2 * edge_index[0], 2 * edge_index[0] + 1]).reshape(
        NC, CH, K)
    dst = edge_index[1].reshape(CH, K)

    def _v(a, w):
        return a.reshape(NC, NP * w // 128, 128)

    z0, r0 = _tc_pre(x, Wl0, Wr0, bl0)
    s0, degp = _sc_agg(HH // 2, True)(z0.reshape(2 * NN, HH // 2), src2, dst)
    z1, r1, invd = _tc_mid1(_v(s0, HH // 2), _v(degp, DW), r0,
                            g1, b1, Wl1, Wr1, bl1)
    s1 = _sc_agg(HH // 2)(z1.reshape(2 * NN, HH // 2), src2, dst)
    z2, r2 = _tc_mid2(_v(s1, HH // 2), r1, invd, g2, b2, Wl2, Wr2, bl2)
    s2 = _sc_agg(PP // 2)(z2.reshape(2 * NN, PP // 2), src2, dst)
    return _tc_fin(_v(s2, PP // 2), r2, invd)
